# baseline (speedup 1.0000x reference)
import functools

import jax
import jax.numpy as jnp
from jax import lax
from jax.experimental import pallas as pl
from jax.experimental.pallas import tpu as pltpu
from jax.experimental.pallas import tpu_sc as plsc

BATCH = 16384
DIM = 256
HALF = 128
LANES = 16
NC = 2
NS = 16
NW = NC * NS
PER_W = BATCH // NW
CHUNK = 64
NCHUNK = PER_W // CHUNK


def _score_one(t, hb, rb, tb, lane):
    acc = jnp.zeros((LANES,), jnp.float32)
    for k in range(HALF // LANES):
        lo = k * LANES
        rh = hb[t, pl.ds(lo, LANES)]
        ih = hb[t, pl.ds(HALF + lo, LANES)]
        rr = rb[t, pl.ds(lo, LANES)]
        ir = rb[t, pl.ds(HALF + lo, LANES)]
        rt = tb[t, pl.ds(lo, LANES)]
        it = tb[t, pl.ds(HALF + lo, LANES)]
        re_s = rr * rt + ir * it
        im_s = rr * it - ir * rt
        acc = acc + rh * re_s + ih * im_s
    for m in (8, 4, 2, 1):
        acc = acc + acc.at[lane ^ m].get(mode="promise_in_bounds")
    return acc


def _make_kernel():
    mesh = plsc.VectorSubcoreMesh(core_axis_name="c", subcore_axis_name="s")

    @functools.partial(
        pl.kernel,
        mesh=mesh,
        out_type=jax.ShapeDtypeStruct((NW, PER_W), jnp.float32),
        scratch_types=[
            pltpu.VMEM((3, NCHUNK, CHUNK), jnp.int32),
            pltpu.VMEM((CHUNK, DIM), jnp.float32),
            pltpu.VMEM((CHUNK, DIM), jnp.float32),
            pltpu.VMEM((CHUNK, DIM), jnp.float32),
            pltpu.VMEM((CHUNK, DIM), jnp.float32),
            pltpu.VMEM((CHUNK, DIM), jnp.float32),
            pltpu.VMEM((CHUNK, DIM), jnp.float32),
            pltpu.VMEM((PER_W,), jnp.float32),
            pltpu.VMEM((CHUNK * LANES,), jnp.float32),
            pltpu.SemaphoreType.DMA,
            pltpu.SemaphoreType.DMA,
        ],
    )
    def compl_ex_sc(idx_hbm, ent_hbm, rel_hbm, out_hbm,
                    idx_v, h0, r0, t0, h1, r1, t1, out_v, acc_v, sem0, sem1):
        wid = lax.axis_index("s") * NC + lax.axis_index("c")
        lane = lax.iota(jnp.int32, LANES)
        hbufs = (h0, h1)
        rbufs = (r0, r1)
        tbufs = (t0, t1)
        sems = (sem0, sem1)

        pltpu.sync_copy(idx_hbm.at[wid], idx_v)

        def fire(c):
            s = sems[c % 2]
            return (
                pltpu.async_copy(ent_hbm.at[idx_v.at[0, c]], hbufs[c % 2], s),
                pltpu.async_copy(rel_hbm.at[idx_v.at[1, c]], rbufs[c % 2], s),
                pltpu.async_copy(ent_hbm.at[idx_v.at[2, c]], tbufs[c % 2], s),
            )

        inflight = fire(0)
        for c in range(NCHUNK):
            nxt = fire(c + 1) if c + 1 < NCHUNK else None
            for d in inflight:
                d.wait()
            inflight = nxt
            hb, rb, tb = hbufs[c % 2], rbufs[c % 2], tbufs[c % 2]

            def body(t):
                s = _score_one(t, hb, rb, tb, lane)
                acc_v[pl.ds(pl.multiple_of(t * LANES, LANES), LANES)] = s

            plsc.parallel_loop(0, CHUNK, 1, unroll=4)(body)

            for g in range(CHUNK // LANES):
                res = jnp.zeros((LANES,), jnp.float32)
                for j in range(LANES):
                    v = acc_v[pl.ds((g * LANES + j) * LANES, LANES)]
                    res = jnp.where(lane == j, v, res)
                out_v[pl.ds(c * CHUNK + g * LANES, LANES)] = res

        pltpu.sync_copy(out_v, out_hbm.at[wid])

    return compl_ex_sc


_compl_ex = _make_kernel()


def kernel(triplet_idx, entity_embedding, relation_embedding):
    idx = triplet_idx.reshape(BATCH, 3).astype(jnp.int32)
    idx = idx.T.reshape(3, NW, NCHUNK, CHUNK).transpose(1, 0, 2, 3)
    out = _compl_ex(idx, entity_embedding, relation_embedding)
    return out.reshape(BATCH, 1)


# speedup vs baseline: 1.0693x; 1.0693x over previous
import functools

import jax
import jax.numpy as jnp
from jax import lax
from jax.experimental import pallas as pl
from jax.experimental.pallas import tpu as pltpu
from jax.experimental.pallas import tpu_sc as plsc

BATCH = 16384
DIM = 256
HALF = 128
LANES = 16
NC = 2
NS = 16
NW = NC * NS
PER_W = BATCH // NW
CHUNK = 64
NCHUNK = PER_W // CHUNK


def _score_one(t, hb, rb, tb, lane):
    acc = jnp.zeros((LANES,), jnp.float32)
    for k in range(HALF // LANES):
        lo = k * LANES
        rh = hb[t, pl.ds(lo, LANES)]
        ih = hb[t, pl.ds(HALF + lo, LANES)]
        rr = rb[t, pl.ds(lo, LANES)]
        ir = rb[t, pl.ds(HALF + lo, LANES)]
        rt = tb[t, pl.ds(lo, LANES)]
        it = tb[t, pl.ds(HALF + lo, LANES)]
        re_s = rr * rt + ir * it
        im_s = rr * it - ir * rt
        acc = acc + rh * re_s + ih * im_s
    for m in (8, 4, 2, 1):
        acc = acc + acc.at[lane ^ m].get(mode="promise_in_bounds")
    return acc


def _make_kernel():
    mesh = plsc.VectorSubcoreMesh(core_axis_name="c", subcore_axis_name="s")

    @functools.partial(
        pl.kernel,
        mesh=mesh,
        out_type=jax.ShapeDtypeStruct((NW, PER_W), jnp.float32),
        scratch_types=[
            pltpu.VMEM((3, NCHUNK, CHUNK), jnp.int32),
            pltpu.VMEM((CHUNK, DIM), jnp.float32),
            pltpu.VMEM((CHUNK, DIM), jnp.float32),
            pltpu.VMEM((CHUNK, DIM), jnp.float32),
            pltpu.VMEM((CHUNK, DIM), jnp.float32),
            pltpu.VMEM((CHUNK, DIM), jnp.float32),
            pltpu.VMEM((CHUNK, DIM), jnp.float32),
            pltpu.VMEM((PER_W,), jnp.float32),
            pltpu.VMEM((CHUNK * LANES,), jnp.float32),
            pltpu.SemaphoreType.DMA,
            pltpu.SemaphoreType.DMA,
        ],
    )
    def compl_ex_sc(idx_hbm, ent_hbm, rel_hbm, out_hbm,
                    idx_v, h0, r0, t0, h1, r1, t1, out_v, acc_v, sem0, sem1):
        wid = lax.axis_index("s") * NC + lax.axis_index("c")
        lane = lax.iota(jnp.int32, LANES)
        hbufs = (h0, h1)
        rbufs = (r0, r1)
        tbufs = (t0, t1)
        sems = (sem0, sem1)

        pltpu.sync_copy(idx_hbm.at[wid], idx_v)

        def fire(c):
            s = sems[c % 2]
            return (
                pltpu.async_copy(ent_hbm.at[idx_v.at[0, c]], hbufs[c % 2], s),
                pltpu.async_copy(rel_hbm.at[idx_v.at[1, c]], rbufs[c % 2], s),
                pltpu.async_copy(ent_hbm.at[idx_v.at[2, c]], tbufs[c % 2], s),
            )

        inflight = fire(0)
        for c in range(NCHUNK):
            nxt = fire(c + 1) if c + 1 < NCHUNK else None
            for d in inflight:
                d.wait()
            inflight = nxt
            hb, rb, tb = hbufs[c % 2], rbufs[c % 2], tbufs[c % 2]

            def body(t):
                s = _score_one(t, hb, rb, tb, lane)
                acc_v[pl.ds(pl.multiple_of(t * LANES, LANES), LANES)] = s

            plsc.parallel_loop(0, CHUNK, 1, unroll=1)(body)

            for g in range(CHUNK // LANES):
                res = jnp.zeros((LANES,), jnp.float32)
                for j in range(LANES):
                    v = acc_v[pl.ds((g * LANES + j) * LANES, LANES)]
                    res = jnp.where(lane == j, v, res)
                out_v[pl.ds(c * CHUNK + g * LANES, LANES)] = res

        pltpu.sync_copy(out_v, out_hbm.at[wid])

    return compl_ex_sc


_compl_ex = _make_kernel()


def kernel(triplet_idx, entity_embedding, relation_embedding):
    idx = triplet_idx.reshape(BATCH, 3).astype(jnp.int32)
    idx = idx.T.reshape(3, NW, NCHUNK, CHUNK).transpose(1, 0, 2, 3)
    out = _compl_ex(idx, entity_embedding, relation_embedding)
    return out.reshape(BATCH, 1)


# speedup vs baseline: 1.2769x; 1.1942x over previous
import functools

import jax
import jax.numpy as jnp
from jax import lax
from jax.experimental import pallas as pl
from jax.experimental.pallas import tpu as pltpu
from jax.experimental.pallas import tpu_sc as plsc

BATCH = 16384
DIM = 256
HALF = 128
LANES = 16
NC = 2
NS = 16
NW = NC * NS
PER_W = BATCH // NW
CHUNK = 32
NCHUNK = PER_W // CHUNK
NBUF = 4
DEPTH = 2


def _score_acc(t, hb, rb, tb):
    acc = jnp.zeros((LANES,), jnp.float32)
    for k in range(HALF // LANES):
        lo = k * LANES
        rh = hb[t, pl.ds(lo, LANES)]
        ih = hb[t, pl.ds(HALF + lo, LANES)]
        rr = rb[t, pl.ds(lo, LANES)]
        ir = rb[t, pl.ds(HALF + lo, LANES)]
        rt = tb[t, pl.ds(lo, LANES)]
        it = tb[t, pl.ds(HALF + lo, LANES)]
        re_s = rr * rt + ir * it
        im_s = rr * it - ir * rt
        acc = acc + rh * re_s + ih * im_s
    return acc


def _make_kernel():
    mesh = plsc.VectorSubcoreMesh(core_axis_name="c", subcore_axis_name="s")

    @functools.partial(
        pl.kernel,
        mesh=mesh,
        out_type=jax.ShapeDtypeStruct((NW, PER_W), jnp.float32),
        scratch_types=[
            pltpu.VMEM((3, NCHUNK, CHUNK), jnp.int32),
            pltpu.VMEM((NBUF * CHUNK, DIM), jnp.float32),
            pltpu.VMEM((NBUF * CHUNK, DIM), jnp.float32),
            pltpu.VMEM((NBUF * CHUNK, DIM), jnp.float32),
            pltpu.VMEM((PER_W,), jnp.float32),
            pltpu.VMEM((CHUNK * LANES,), jnp.float32),
            pltpu.SemaphoreType.DMA,
            pltpu.SemaphoreType.DMA,
            pltpu.SemaphoreType.DMA,
            pltpu.SemaphoreType.DMA,
        ],
    )
    def compl_ex_sc(idx_hbm, ent_hbm, rel_hbm, out_hbm,
                    idx_v, hbuf, rbuf, tbuf, out_v, acc_v, s0, s1, s2, s3):
        wid = lax.axis_index("s") * NC + lax.axis_index("c")
        lane = lax.iota(jnp.int32, LANES)
        sems = (s0, s1, s2, s3)

        pltpu.sync_copy(idx_hbm.at[wid], idx_v)

        def bufs(b):
            sl = pl.ds(b * CHUNK, CHUNK)
            return hbuf.at[sl], rbuf.at[sl], tbuf.at[sl]

        def fire(c, b):
            hb, rb, tb = bufs(b)
            pltpu.async_copy(ent_hbm.at[idx_v.at[0, c]], hb, sems[b])
            pltpu.async_copy(rel_hbm.at[idx_v.at[1, c]], rb, sems[b])
            pltpu.async_copy(ent_hbm.at[idx_v.at[2, c]], tb, sems[b])

        def drain(c, b):
            hb, rb, tb = bufs(b)
            pltpu.make_async_copy(
                ent_hbm.at[idx_v.at[0, c]], hb, sems[b]).wait()
            pltpu.make_async_copy(
                rel_hbm.at[idx_v.at[1, c]], rb, sems[b]).wait()
            pltpu.make_async_copy(
                ent_hbm.at[idx_v.at[2, c]], tb, sems[b]).wait()

        for p in range(DEPTH):
            fire(p, p)

        def chunk_body(c, carry):
            parity = c & (NBUF - 1)
            nxt = c + DEPTH < NCHUNK

            for p in range(NBUF):

                @pl.when(nxt & (parity == p))
                def _(p=p):
                    fire(c + DEPTH, (p + DEPTH) & (NBUF - 1))

            for p in range(NBUF):

                @pl.when(parity == p)
                def _(p=p):
                    drain(c, p)

            boff = pl.multiple_of(parity * CHUNK, CHUNK)

            def body(t):
                acc = _score_acc(boff + t, hbuf, rbuf, tbuf)
                for m in (8, 4, 2, 1):
                    acc = acc + acc.at[lane ^ m].get(mode="promise_in_bounds")
                acc_v[pl.ds(pl.multiple_of(t * LANES, LANES), LANES)] = acc

            plsc.parallel_loop(0, CHUNK, 1, unroll=2)(body)

            for g in range(CHUNK // LANES):
                res = jnp.zeros((LANES,), jnp.float32)
                for j in range(LANES):
                    v = acc_v[pl.ds((g * LANES + j) * LANES, LANES)]
                    res = jnp.where(lane == j, v, res)
                base = pl.multiple_of(c * CHUNK, CHUNK) + g * LANES
                out_v[pl.ds(base, LANES)] = res

            return carry

        lax.fori_loop(0, NCHUNK, chunk_body, 0)
        pltpu.sync_copy(out_v, out_hbm.at[wid])

    return compl_ex_sc


_compl_ex = _make_kernel()


def kernel(triplet_idx, entity_embedding, relation_embedding):
    idx = triplet_idx.reshape(BATCH, 3).astype(jnp.int32)
    idx = idx.T.reshape(3, NW, NCHUNK, CHUNK).transpose(1, 0, 2, 3)
    out = _compl_ex(idx, entity_embedding, relation_embedding)
    return out.reshape(BATCH, 1)
